# baseline (speedup 1.0000x reference)
import functools

import jax
import jax.numpy as jnp
from jax import lax
from jax.experimental import pallas as pl
from jax.experimental.pallas import tpu as pltpu
from jax.experimental.pallas import tpu_sc as plsc

_V = 1000000
_D = 64
_B = 16384

_info = plsc.get_sparse_core_info()
_NC = _info.num_cores
_NS = _info.num_subcores
_NW = _NC * _NS
_BPW = _B // _NW
_CHUNK = 128
_NCHUNK = _BPW // _CHUNK
_HALF = 2 * _CHUNK


def _glove_kernel(left_hbm, right_hbm, l_emb_hbm, l_bias_hbm, r_emb_hbm,
                  r_bias_hbm, out_hbm,
                  lidx_v, ridx_v, lidx2_v, ridx2_v, lpar_v, rpar_v,
                  lrows_v, rrows_v, lb_v, rb_v, out_v, sem, bsem):
    wid = lax.axis_index("s") * _NC + lax.axis_index("c")
    base = wid * _BPW

    pltpu.sync_copy(left_hbm.at[wid], lidx_v)
    pltpu.sync_copy(right_hbm.at[wid], ridx_v)

    iota16 = lax.iota(jnp.int32, 16)

    for j in range(_NCHUNK):
        for t in range(_CHUNK // 16):
            s = pl.ds(t * 16, 16)
            li = lidx_v[j, s]
            ri = ridx_v[j, s]
            lidx2_v[j, s] = li >> 1
            ridx2_v[j, s] = ri >> 1
            lpar_v[pl.ds(j * _CHUNK + t * 16, 16)] = li & 1
            rpar_v[pl.ds(j * _CHUNK + t * 16, 16)] = ri & 1

    bias_copies = []
    for j in range(_NCHUNK):
        dst = pl.ds(j * _CHUNK, _CHUNK)
        bias_copies.append(pltpu.async_copy(
            l_bias_hbm.at[lidx_v.at[j]], lb_v.at[dst], bsem))
        bias_copies.append(pltpu.async_copy(
            r_bias_hbm.at[ridx_v.at[j]], rb_v.at[dst], bsem))

    def do_half(h):
        copies = []
        for jj in range(2):
            j = 2 * h + jj
            dst = pl.ds(jj * _CHUNK, _CHUNK)
            copies.append(pltpu.async_copy(
                l_emb_hbm.at[lidx2_v.at[j]], lrows_v.at[dst], sem))
            copies.append(pltpu.async_copy(
                r_emb_hbm.at[ridx2_v.at[j]], rrows_v.at[dst], sem))
        for c in copies:
            c.wait()

        def group_body(g, _):
            p0 = h * _HALF + g * 16
            par_l = lpar_v[pl.ds(p0, 16)]
            par_r = rpar_v[pl.ds(p0, 16)]
            dots = jnp.zeros((16,), jnp.float32)
            for k in range(16):
                row = g * 16 + k
                sl = jnp.sum(jnp.where(iota16 == k, par_l, 0))
                sr = jnp.sum(jnp.where(iota16 == k, par_r, 0))
                acc = jnp.zeros((16,), jnp.float32)
                for c in range(_D // 16):
                    llo = lrows_v[row, pl.ds(c * 16, 16)]
                    lhi = lrows_v[row, pl.ds(64 + c * 16, 16)]
                    rlo = rrows_v[row, pl.ds(c * 16, 16)]
                    rhi = rrows_v[row, pl.ds(64 + c * 16, 16)]
                    lv = jnp.where(sl == 1, lhi, llo)
                    rv = jnp.where(sr == 1, rhi, rlo)
                    acc = acc + lv * rv
                dots = jnp.where(iota16 == k, jnp.sum(acc), dots)
            out_v[pl.ds(p0, 16)] = dots
            return 0

        lax.fori_loop(0, _HALF // 16, group_body, 0)

    for h in range(_BPW // _HALF):
        do_half(h)

    for c in bias_copies:
        c.wait()
    for t in range(_BPW // 16):
        s = pl.ds(t * 16, 16)
        out_v[s] = out_v[s] + lb_v[s] + rb_v[s]

    pltpu.sync_copy(out_v, out_hbm.at[pl.ds(base, _BPW)])


@functools.partial(jax.jit, donate_argnums=())
def kernel(left, right, l_emb, l_bias, r_emb, r_bias):
    mesh = plsc.VectorSubcoreMesh(core_axis_name="c", subcore_axis_name="s")
    left_r = left.reshape(_NW, _NCHUNK, _CHUNK)
    right_r = right.reshape(_NW, _NCHUNK, _CHUNK)
    l_emb2 = l_emb.reshape(_V // 2, 2 * _D)
    r_emb2 = r_emb.reshape(_V // 2, 2 * _D)
    l_bias_f = l_bias.reshape(_V)
    r_bias_f = r_bias.reshape(_V)
    run = pl.kernel(
        _glove_kernel,
        mesh=mesh,
        out_type=jax.ShapeDtypeStruct((_B,), jnp.float32),
        compiler_params=pltpu.CompilerParams(
            needs_layout_passes=False, use_tc_tiling_on_sc=True),
        scratch_types=[
            pltpu.VMEM((_NCHUNK, _CHUNK), jnp.int32),
            pltpu.VMEM((_NCHUNK, _CHUNK), jnp.int32),
            pltpu.VMEM((_NCHUNK, _CHUNK), jnp.int32),
            pltpu.VMEM((_NCHUNK, _CHUNK), jnp.int32),
            pltpu.VMEM((_BPW,), jnp.int32),
            pltpu.VMEM((_BPW,), jnp.int32),
            pltpu.VMEM((_HALF, 2 * _D), jnp.float32),
            pltpu.VMEM((_HALF, 2 * _D), jnp.float32),
            pltpu.VMEM((_BPW,), jnp.float32),
            pltpu.VMEM((_BPW,), jnp.float32),
            pltpu.VMEM((_BPW,), jnp.float32),
            pltpu.SemaphoreType.DMA,
            pltpu.SemaphoreType.DMA,
        ],
    )
    return run(left_r, right_r, l_emb2, l_bias_f, r_emb2, r_bias_f)


# speedup vs baseline: 2.3169x; 2.3169x over previous
import functools

import jax
import jax.numpy as jnp
from jax import lax
from jax.experimental import pallas as pl
from jax.experimental.pallas import tpu as pltpu
from jax.experimental.pallas import tpu_sc as plsc

_V = 1000000
_D = 64
_B = 16384

_info = plsc.get_sparse_core_info()
_NC = _info.num_cores
_NS = _info.num_subcores
_NW = _NC * _NS
_BPW = _B // _NW
_CHUNK = 128
_NCHUNK = _BPW // _CHUNK

_WIN = 512
_VMAIN = (_V // _WIN) * _WIN
_NWIN = _VMAIN // _WIN
_TAIL = _V - _VMAIN
_WPW_BASE = _NWIN // _NW
_WPW_REM = _NWIN - _WPW_BASE * _NW


def _extract_kernel(tblT_hbm, idx_hbm, rows_hbm,
                    idx_v, mp_v, cbp_v, cbc_v, stage_v, tstage_v, rowbufs_v,
                    stsem, rowsem):
    wid = lax.axis_index("s") * _NC + lax.axis_index("c")
    iota16 = lax.iota(jnp.int32, 16)

    pltpu.sync_copy(idx_hbm, idx_v)

    start = wid * _WPW_BASE + jnp.minimum(wid, _WPW_REM)
    n_my = _WPW_BASE + jnp.where(wid < _WPW_REM, 1, 0)
    lo = start * _WIN
    hi = jnp.where(wid == _NW - 1, _V, (start + n_my) * _WIN)

    def build(v, off):
        c = idx_v[pl.ds(v * 16, 16)]
        m = (c >= lo) & (c < hi)
        plsc.store_compressed(mp_v.at[pl.ds(off, 16)], v * 16 + iota16, mask=m)
        return off + jnp.sum(m.astype(jnp.int32))

    nm = lax.fori_loop(0, _B // 16, build, 0)
    ng = (nm + 15) // 16

    def drain():
        pltpu.make_async_copy(
            rows_hbm.at[pl.ds(0, _D)], rowbufs_v.at[0], rowsem).wait()

    def process_window(lo_w, width, inflight, buf):
        def scan(v, coff):
            valid = (v * 16 + iota16) < nm
            pids = mp_v[pl.ds(v * 16, 16)]
            cs = plsc.load_gather(idx_v, [pids], mask=valid)
            m = valid & (cs >= lo_w) & (cs < lo_w + width)
            plsc.store_compressed(cbp_v.at[pl.ds(coff, 16)], pids, mask=m)
            plsc.store_compressed(cbc_v.at[pl.ds(coff, 16)], cs, mask=m)
            return coff + jnp.sum(m.astype(jnp.int32))

        nc = lax.fori_loop(0, ng, scan, 0)

        def hitgroup(g, infl):
            pids16 = cbp_v[pl.ds(g * 16, 16)]
            offs16 = jnp.clip(cbc_v[pl.ds(g * 16, 16)] - lo_w, 0, width - 1)
            for h in range(16):
                valid = (g * 16 + h) < nc
                p_h = jnp.sum(jnp.where(iota16 == h, pids16, 0))
                off_h = jnp.sum(jnp.where(iota16 == h, offs16, 0))
                offv = jnp.broadcast_to(off_h, (16,))
                do_drain = valid & (infl >= 16)
                pl.when(do_drain)(drain)
                for q in range(4):
                    d = 16 * q + iota16
                    val = plsc.load_gather(buf, [d >> 3, d & 7, offv])
                    rowbufs_v[h, pl.ds(16 * q, 16)] = val

                def fire():
                    pltpu.async_copy(
                        rowbufs_v.at[h], rows_hbm.at[pl.ds(p_h * _D, _D)],
                        rowsem)

                pl.when(valid)(fire)
                infl = (infl + valid.astype(jnp.int32)
                        - do_drain.astype(jnp.int32))
            return infl

        return lax.fori_loop(0, (nc + 15) // 16, hitgroup, inflight)

    def chunk_body(j, inflight):
        lo_w = (start + j) * _WIN
        cb = pl.multiple_of(lo_w, _WIN)
        copies = []
        for dhi in range(8):
            copies.append(pltpu.async_copy(
                tblT_hbm.at[pl.ds(dhi * 8, 8), pl.ds(cb, _WIN)],
                stage_v.at[dhi], stsem))
        for cp in copies:
            cp.wait()
        return process_window(lo_w, _WIN, inflight, stage_v)

    inflight = lax.fori_loop(0, n_my, chunk_body, 0)

    tail_copies = []
    for dhi in range(8):
        tail_copies.append(pltpu.async_copy(
            tblT_hbm.at[pl.ds(dhi * 8, 8), pl.ds(_VMAIN, _TAIL)],
            tstage_v.at[dhi], stsem))
    for cp in tail_copies:
        cp.wait()
    inflight = process_window(_VMAIN, _TAIL, inflight, tstage_v)

    def final_drain(_, x):
        pltpu.make_async_copy(
            rows_hbm.at[pl.ds(0, _D)], rowbufs_v.at[0], rowsem).wait()
        return x

    lax.fori_loop(0, inflight, final_drain, 0)


def _dot_kernel(left_hbm, right_hbm, lrows_hbm, rrows_hbm,
                l_bias_hbm, r_bias_hbm, out_hbm,
                lidx_v, ridx_v, lrows_v, rrows_v, lb_v, rb_v, out_v, sem):
    wid = lax.axis_index("s") * _NC + lax.axis_index("c")
    base = wid * _BPW

    pltpu.sync_copy(left_hbm.at[wid], lidx_v)
    pltpu.sync_copy(right_hbm.at[wid], ridx_v)

    copies = [
        pltpu.async_copy(lrows_hbm.at[pl.ds(base * _D, _BPW * _D)],
                         lrows_v, sem),
        pltpu.async_copy(rrows_hbm.at[pl.ds(base * _D, _BPW * _D)],
                         rrows_v, sem),
    ]
    for j in range(_NCHUNK):
        dst = pl.ds(j * _CHUNK, _CHUNK)
        copies.append(pltpu.async_copy(
            l_bias_hbm.at[lidx_v.at[j]], lb_v.at[dst], sem))
        copies.append(pltpu.async_copy(
            r_bias_hbm.at[ridx_v.at[j]], rb_v.at[dst], sem))
    for c in copies:
        c.wait()

    iota16 = lax.iota(jnp.int32, 16)

    def group_body(g, _):
        i0 = g * 16
        dots = jnp.zeros((16,), jnp.float32)
        for k in range(16):
            i = i0 + k
            acc = (lrows_v[pl.ds(i * _D, 16)] * rrows_v[pl.ds(i * _D, 16)])
            for c in range(1, _D // 16):
                acc = acc + (lrows_v[pl.ds(i * _D + 16 * c, 16)]
                             * rrows_v[pl.ds(i * _D + 16 * c, 16)])
            dots = jnp.where(iota16 == k, jnp.sum(acc), dots)
        out_v[pl.ds(i0, 16)] = dots + lb_v[pl.ds(i0, 16)] + rb_v[pl.ds(i0, 16)]
        return 0

    lax.fori_loop(0, _BPW // 16, group_body, 0)

    pltpu.sync_copy(out_v, out_hbm.at[pl.ds(base, _BPW)])


@functools.partial(jax.jit, donate_argnums=())
def kernel(left, right, l_emb, l_bias, r_emb, r_bias):
    mesh = plsc.VectorSubcoreMesh(core_axis_name="c", subcore_axis_name="s")
    cparams = pltpu.CompilerParams(
        needs_layout_passes=False, use_tc_tiling_on_sc=True)

    extract = pl.kernel(
        _extract_kernel,
        mesh=mesh,
        out_type=jax.ShapeDtypeStruct((_B * _D,), jnp.float32),
        compiler_params=cparams,
        scratch_types=[
            pltpu.VMEM((_B,), jnp.int32),
            pltpu.VMEM((_B + 16,), jnp.int32),
            pltpu.VMEM((_B + 16,), jnp.int32),
            pltpu.VMEM((_B + 16,), jnp.int32),
            pltpu.VMEM((8, 8, _WIN), jnp.float32),
            pltpu.VMEM((8, 8, _TAIL), jnp.float32),
            pltpu.VMEM((16, _D), jnp.float32),
            pltpu.SemaphoreType.DMA,
            pltpu.SemaphoreType.DMA,
        ],
    )

    dot = pl.kernel(
        _dot_kernel,
        mesh=mesh,
        out_type=jax.ShapeDtypeStruct((_B,), jnp.float32),
        compiler_params=cparams,
        scratch_types=[
            pltpu.VMEM((_NCHUNK, _CHUNK), jnp.int32),
            pltpu.VMEM((_NCHUNK, _CHUNK), jnp.int32),
            pltpu.VMEM((_BPW * _D,), jnp.float32),
            pltpu.VMEM((_BPW * _D,), jnp.float32),
            pltpu.VMEM((_BPW,), jnp.float32),
            pltpu.VMEM((_BPW,), jnp.float32),
            pltpu.VMEM((_BPW,), jnp.float32),
            pltpu.SemaphoreType.DMA,
        ],
    )

    left_r = left.reshape(_NW, _NCHUNK, _CHUNK)
    right_r = right.reshape(_NW, _NCHUNK, _CHUNK)
    l_bias_f = l_bias.reshape(_V)
    r_bias_f = r_bias.reshape(_V)

    rows_l = extract(l_emb.T, left)
    rows_r = extract(r_emb.T, right)
    return dot(left_r, right_r, rows_l, rows_r, l_bias_f, r_bias_f)


# speedup vs baseline: 3.7933x; 1.6372x over previous
import functools

import jax
import jax.numpy as jnp
from jax import lax
from jax.experimental import pallas as pl
from jax.experimental.pallas import tpu as pltpu
from jax.experimental.pallas import tpu_sc as plsc

_V = 1000000
_D = 64
_B = 16384

_info = plsc.get_sparse_core_info()
_NC = _info.num_cores
_NS = _info.num_subcores
_NW = _NC * _NS
_BPW = _B // _NW
_CHUNK = 128
_NCHUNK = _BPW // _CHUNK

_WIN = 512
_VMAIN = (_V // _WIN) * _WIN
_NWIN = _VMAIN // _WIN
_TAIL = _V - _VMAIN
_WPW_BASE = _NWIN // _NW
_WPW_REM = _NWIN - _WPW_BASE * _NW


_NWFIX = _WPW_BASE + 1


def _extract_kernel(tblT_hbm, idx_hbm, rows_hbm,
                    idx_v, mp_v, cbp_v, stage0_v, stage1_v, tstage_v,
                    rowbufs_v, sem0, sem1, rowsem):
    wid = lax.axis_index("s") * _NC + lax.axis_index("c")
    iota16 = lax.iota(jnp.int32, 16)

    pltpu.sync_copy(idx_hbm, idx_v)

    start = wid * _WPW_BASE + jnp.minimum(wid, _WPW_REM)
    n_my = _WPW_BASE + jnp.where(wid < _WPW_REM, 1, 0)
    lo = start * _WIN
    hi = jnp.where(wid == _NW - 1, _V, (start + n_my) * _WIN)

    def build(v, off):
        c = idx_v[pl.ds(v * 16, 16)]
        m = (c >= lo) & (c < hi)
        plsc.store_compressed(mp_v.at[pl.ds(off, 16)], v * 16 + iota16, mask=m)
        return off + jnp.sum(m.astype(jnp.int32))

    nm = lax.fori_loop(0, _B // 16, build, 0)
    ng = (nm + 15) // 16

    def drain():
        pltpu.make_async_copy(
            rows_hbm.at[pl.ds(0, _D)], rowbufs_v.at[0], rowsem).wait()

    def win_lo(j):
        return jnp.minimum((start + j) * _WIN, _VMAIN - _WIN)

    def fire_stage(j, buf, sem):
        cb = pl.multiple_of(win_lo(j), _WIN)
        for dhi in range(8):
            pltpu.async_copy(
                tblT_hbm.at[pl.ds(dhi * 8, 8), pl.ds(cb, _WIN)],
                buf.at[dhi], sem)

    def wait_stage(buf, sem):
        for dhi in range(8):
            pltpu.make_async_copy(
                tblT_hbm.at[pl.ds(0, 8), pl.ds(0, _WIN)], buf.at[dhi],
                sem).wait()

    def process_window(lo_w, width, inflight, buf):
        def scan(v, coff):
            valid = (v * 16 + iota16) < nm
            pids = mp_v[pl.ds(v * 16, 16)]
            cs = plsc.load_gather(idx_v, [pids], mask=valid)
            m = valid & (cs >= lo_w) & (cs < lo_w + width)
            plsc.store_compressed(cbp_v.at[pl.ds(coff, 16)], pids, mask=m)
            return coff + jnp.sum(m.astype(jnp.int32))

        nc = lax.fori_loop(0, ng, scan, 0)

        def hitgroup(g, infl):
            gvalid = (g * 16 + iota16) < nc
            pids16 = cbp_v[pl.ds(g * 16, 16)]
            cs16 = plsc.load_gather(idx_v, [pids16], mask=gvalid)
            offs16 = jnp.clip(cs16 - lo_w, 0, width - 1)
            for h in range(16):
                valid = (g * 16 + h) < nc
                p_h = jnp.sum(jnp.where(iota16 == h, pids16, 0))
                off_h = jnp.sum(jnp.where(iota16 == h, offs16, 0))
                offv = jnp.broadcast_to(off_h, (16,))
                do_drain = valid & (infl >= 16)
                pl.when(do_drain)(drain)
                for q in range(4):
                    d = 16 * q + iota16
                    val = plsc.load_gather(buf, [d >> 3, d & 7, offv])
                    rowbufs_v[h, pl.ds(16 * q, 16)] = val

                def fire():
                    pltpu.async_copy(
                        rowbufs_v.at[h], rows_hbm.at[pl.ds(p_h * _D, _D)],
                        rowsem)

                pl.when(valid)(fire)
                infl = (infl + valid.astype(jnp.int32)
                        - do_drain.astype(jnp.int32))
            return infl

        return lax.fori_loop(0, (nc + 15) // 16, hitgroup, inflight)

    fire_stage(0, stage0_v, sem0)

    def pipe_body(jj, inflight):
        j0 = 2 * jj
        fire_stage(j0 + 1, stage1_v, sem1)
        wait_stage(stage0_v, sem0)
        inflight = process_window(win_lo(j0), _WIN, inflight, stage0_v)
        fire_stage(j0 + 2, stage0_v, sem0)
        wait_stage(stage1_v, sem1)
        return process_window(win_lo(j0 + 1), _WIN, inflight, stage1_v)

    inflight = lax.fori_loop(0, _NWFIX // 2, pipe_body, 0)
    wait_stage(stage0_v, sem0)

    tail_copies = []
    for dhi in range(8):
        tail_copies.append(pltpu.async_copy(
            tblT_hbm.at[pl.ds(dhi * 8, 8), pl.ds(_VMAIN, _TAIL)],
            tstage_v.at[dhi], sem0))
    for cp in tail_copies:
        cp.wait()
    inflight = process_window(_VMAIN, _TAIL, inflight, tstage_v)

    def final_drain(_, x):
        pltpu.make_async_copy(
            rows_hbm.at[pl.ds(0, _D)], rowbufs_v.at[0], rowsem).wait()
        return x

    lax.fori_loop(0, inflight, final_drain, 0)


def _dot_kernel(left_hbm, right_hbm, lrows_hbm, rrows_hbm,
                l_bias_hbm, r_bias_hbm, out_hbm,
                lidx_v, ridx_v, lrows_v, rrows_v, lb_v, rb_v, out_v, sem):
    wid = lax.axis_index("s") * _NC + lax.axis_index("c")
    base = wid * _BPW

    pltpu.sync_copy(left_hbm.at[wid], lidx_v)
    pltpu.sync_copy(right_hbm.at[wid], ridx_v)

    copies = [
        pltpu.async_copy(lrows_hbm.at[pl.ds(base * _D, _BPW * _D)],
                         lrows_v, sem),
        pltpu.async_copy(rrows_hbm.at[pl.ds(base * _D, _BPW * _D)],
                         rrows_v, sem),
    ]
    for j in range(_NCHUNK):
        dst = pl.ds(j * _CHUNK, _CHUNK)
        copies.append(pltpu.async_copy(
            l_bias_hbm.at[lidx_v.at[j]], lb_v.at[dst], sem))
        copies.append(pltpu.async_copy(
            r_bias_hbm.at[ridx_v.at[j]], rb_v.at[dst], sem))
    for c in copies:
        c.wait()

    iota16 = lax.iota(jnp.int32, 16)

    def group_body(g, _):
        i0 = g * 16
        dots = jnp.zeros((16,), jnp.float32)
        for k in range(16):
            i = i0 + k
            acc = (lrows_v[pl.ds(i * _D, 16)] * rrows_v[pl.ds(i * _D, 16)])
            for c in range(1, _D // 16):
                acc = acc + (lrows_v[pl.ds(i * _D + 16 * c, 16)]
                             * rrows_v[pl.ds(i * _D + 16 * c, 16)])
            dots = jnp.where(iota16 == k, jnp.sum(acc), dots)
        out_v[pl.ds(i0, 16)] = dots + lb_v[pl.ds(i0, 16)] + rb_v[pl.ds(i0, 16)]
        return 0

    lax.fori_loop(0, _BPW // 16, group_body, 0)

    pltpu.sync_copy(out_v, out_hbm.at[pl.ds(base, _BPW)])


@functools.partial(jax.jit, donate_argnums=())
def kernel(left, right, l_emb, l_bias, r_emb, r_bias):
    mesh = plsc.VectorSubcoreMesh(core_axis_name="c", subcore_axis_name="s")
    cparams = pltpu.CompilerParams(
        needs_layout_passes=False, use_tc_tiling_on_sc=True)

    extract = pl.kernel(
        _extract_kernel,
        mesh=mesh,
        out_type=jax.ShapeDtypeStruct((_B * _D,), jnp.float32),
        compiler_params=cparams,
        scratch_types=[
            pltpu.VMEM((_B,), jnp.int32),
            pltpu.VMEM((_B + 16,), jnp.int32),
            pltpu.VMEM((_B + 16,), jnp.int32),
            pltpu.VMEM((8, 8, _WIN), jnp.float32),
            pltpu.VMEM((8, 8, _WIN), jnp.float32),
            pltpu.VMEM((8, 8, _TAIL), jnp.float32),
            pltpu.VMEM((16, _D), jnp.float32),
            pltpu.SemaphoreType.DMA,
            pltpu.SemaphoreType.DMA,
            pltpu.SemaphoreType.DMA,
        ],
    )

    dot = pl.kernel(
        _dot_kernel,
        mesh=mesh,
        out_type=jax.ShapeDtypeStruct((_B,), jnp.float32),
        compiler_params=cparams,
        scratch_types=[
            pltpu.VMEM((_NCHUNK, _CHUNK), jnp.int32),
            pltpu.VMEM((_NCHUNK, _CHUNK), jnp.int32),
            pltpu.VMEM((_BPW * _D,), jnp.float32),
            pltpu.VMEM((_BPW * _D,), jnp.float32),
            pltpu.VMEM((_BPW,), jnp.float32),
            pltpu.VMEM((_BPW,), jnp.float32),
            pltpu.VMEM((_BPW,), jnp.float32),
            pltpu.SemaphoreType.DMA,
        ],
    )

    left_r = left.reshape(_NW, _NCHUNK, _CHUNK)
    right_r = right.reshape(_NW, _NCHUNK, _CHUNK)
    l_bias_f = l_bias.reshape(_V)
    r_bias_f = r_bias.reshape(_V)

    rows_l = extract(l_emb.T, left)
    rows_r = extract(r_emb.T, right)
    return dot(left_r, right_r, rows_l, rows_r, l_bias_f, r_bias_f)
